# baseline (speedup 1.0000x reference)
import functools

import jax
import jax.numpy as jnp
from jax import lax
from jax.experimental import pallas as pl
from jax.experimental.pallas import tpu as pltpu
from jax.experimental.pallas import tpu_sc as plsc

N_NODES = 10000
N_PAD = 10240
N_EDGES = 320000
D = 128
EPS = 1e-5

NC = 2
NS = 16
NW = NC * NS
E_CHK = 80
N_CHK = 128
TOTC = NW * N_CHK
E_TOT = TOTC * E_CHK
K0 = 200
K1 = 56
ROWS_PER_SUB = N_PAD // NS
NBUF = 4
NDST = 8


def _fill_vmem(ref, nrows, value):
    def body(i, _):
        for j in range(D // 16):
            ref[i, pl.ds(j * 16, 16)] = jnp.full((16,), value, jnp.float32)
        return 0
    lax.fori_loop(0, nrows, body, 0)


def _zero_share(slab, sid, shared):
    def zero_slab(k, _):
        r0 = sid * ROWS_PER_SUB + k * E_CHK
        pltpu.sync_copy(slab, shared.at[pl.ds(r0, E_CHK)])
        return 0
    lax.fori_loop(0, ROWS_PER_SUB // E_CHK, zero_slab, 0)


def _copy_share_out(slab, cid, sid, shared, out_hbm):
    def copy_slab(k, _):
        r0 = sid * ROWS_PER_SUB + k * E_CHK
        pltpu.sync_copy(shared.at[pl.ds(r0, E_CHK)], slab)
        pltpu.sync_copy(slab, out_hbm.at[cid, pl.ds(r0, E_CHK)])
        return 0
    lax.fori_loop(0, ROWS_PER_SUB // E_CHK, copy_slab, 0)


def _sc_agg_body(x_hbm, src_hbm, w_hbm, dst_hbm, agg_out,
                 ssliv_v, wsliv_v, dsliv_v, rows_v, acc_sh,
                 gsem, ssem, lsem, dsem):
    cid = lax.axis_index("c")
    sid = lax.axis_index("s")
    nchk = jnp.where(cid == 0, K0, K1)
    base = jnp.where(cid == 0, sid * K0, NS * K0 + sid * K1)

    _fill_vmem(rows_v.at[0], E_CHK, 0.0)
    _zero_share(rows_v.at[0], sid, acc_sh)
    plsc.subcore_barrier()

    def issue_sw(c, slot):
        pltpu.async_copy(src_hbm.at[base + c], ssliv_v.at[slot],
                         lsem.at[slot])
        pltpu.async_copy(w_hbm.at[base + c], wsliv_v.at[slot], lsem.at[slot])

    def issue_dst(c, slot):
        pltpu.async_copy(dst_hbm.at[base + c], dsliv_v.at[slot],
                         dsem.at[slot])

    def wait_sw(c, slot):
        pltpu.make_async_copy(src_hbm.at[base + c], ssliv_v.at[slot],
                              lsem.at[slot]).wait()
        pltpu.make_async_copy(w_hbm.at[base + c], wsliv_v.at[slot],
                              lsem.at[slot]).wait()

    def wait_dst(c, slot):
        pltpu.make_async_copy(dst_hbm.at[base + c], dsliv_v.at[slot],
                              dsem.at[slot]).wait()

    def issue_gather(c, slot):
        pltpu.async_copy(x_hbm.at[ssliv_v.at[slot]], rows_v.at[slot],
                         gsem.at[slot])

    def wait_gather(c, slot):
        pltpu.make_async_copy(x_hbm.at[ssliv_v.at[slot]],
                              rows_v.at[slot], gsem.at[slot]).wait()

    def issue_scatter(c, slot, dslot):
        pltpu.async_copy(rows_v.at[slot], acc_sh.at[dsliv_v.at[dslot]],
                         ssem.at[slot], add=True)

    def wait_scatter(c, slot, dslot):
        pltpu.make_async_copy(rows_v.at[slot], acc_sh.at[dsliv_v.at[dslot]],
                              ssem.at[slot]).wait()

    for k in range(NDST - 2):
        issue_dst(k, k)
    for k in range(NBUF):
        issue_sw(k, k)
    for k in range(2):
        wait_sw(k, k)
        issue_gather(k, k)

    def step(c, _):
        r = lax.rem(c, NBUF)
        dslot = lax.rem(c, NDST)

        @pl.when(c >= 2)
        def _():
            wait_scatter(c - 2, lax.rem(c - 2, NBUF), lax.rem(c - 2, NDST))

        @pl.when(c + NDST - 2 < nchk)
        def _():
            issue_dst(c + NDST - 2, lax.rem(c + NDST - 2, NDST))

        @pl.when(c + 2 < nchk)
        def _():
            wait_sw(c + 2, lax.rem(c + 2, NBUF))
            issue_gather(c + 2, lax.rem(c + 2, NBUF))

        wait_gather(c, r)

        def scale(k, _):
            ww = wsliv_v[r, pl.ds(k * 16, 16)]
            for l in range(16):
                w = ww[l]
                e = k * 16 + l
                for j in range(D // 16):
                    v = rows_v[r, e, pl.ds(j * 16, 16)]
                    rows_v[r, e, pl.ds(j * 16, 16)] = v * w
            return 0
        lax.fori_loop(0, E_CHK // 16, scale, 0)

        @pl.when(c + NBUF < nchk)
        def _():
            issue_sw(c + NBUF, r)

        wait_dst(c, dslot)
        issue_scatter(c, r, dslot)
        return 0
    lax.fori_loop(0, nchk, step, 0)
    wait_scatter(nchk - 2, 2, NDST - 2)
    wait_scatter(nchk - 1, 3, NDST - 1)
    plsc.subcore_barrier()

    _copy_share_out(rows_v.at[0], cid, sid, acc_sh, agg_out)


def _make_sc_agg():
    mesh = plsc.VectorSubcoreMesh(core_axis_name="c", subcore_axis_name="s")
    scratch = [
        pltpu.VMEM((NBUF, E_CHK), jnp.int32),
        pltpu.VMEM((NBUF, E_CHK), jnp.float32),
        pltpu.VMEM((NDST, E_CHK), jnp.int32),
        pltpu.VMEM((NBUF, E_CHK, D), jnp.float32),
        pltpu.VMEM_SHARED((N_PAD, D), jnp.float32),
        pltpu.SemaphoreType.DMA((NBUF,)),
        pltpu.SemaphoreType.DMA((NBUF,)),
        pltpu.SemaphoreType.DMA((NBUF,)),
        pltpu.SemaphoreType.DMA((NDST,)),
    ]
    return pl.kernel(
        _sc_agg_body,
        out_type=jax.ShapeDtypeStruct((NC, N_PAD, D), jnp.float32),
        mesh=mesh, scratch_types=scratch, name="sc_agg")


def _sc_cnt_body(dst_hbm, cnt_out, dst_v, ones_v, zbuf_v, cnt_sh):
    cid = lax.axis_index("c")
    sid = lax.axis_index("s")
    wid = cid * NS + sid

    _fill_vmem(zbuf_v, E_CHK, 0.0)
    _fill_vmem(ones_v, E_CHK, 1.0)
    _zero_share(zbuf_v, sid, cnt_sh)
    plsc.subcore_barrier()

    pltpu.sync_copy(dst_hbm.at[pl.ds(wid * N_CHK, N_CHK)], dst_v)

    def chunk(c, _):
        pltpu.sync_copy(ones_v, cnt_sh.at[dst_v.at[c]], add=True)
        return 0
    lax.fori_loop(0, N_CHK, chunk, 0)
    plsc.subcore_barrier()

    _copy_share_out(zbuf_v, cid, sid, cnt_sh, cnt_out)


def _make_sc_cnt():
    mesh = plsc.VectorSubcoreMesh(core_axis_name="c", subcore_axis_name="s")
    scratch = [
        pltpu.VMEM((N_CHK, E_CHK), jnp.int32),
        pltpu.VMEM((E_CHK, D), jnp.float32),
        pltpu.VMEM((E_CHK, D), jnp.float32),
        pltpu.VMEM_SHARED((N_PAD, D), jnp.float32),
    ]
    return pl.kernel(
        _sc_cnt_body,
        out_type=jax.ShapeDtypeStruct((NC, N_PAD, D), jnp.float32),
        mesh=mesh, scratch_types=scratch, name="sc_cnt")


_sc_agg = _make_sc_agg()
_sc_cnt = _make_sc_cnt()


def _mean_div(agg_ref, cnt_ref):
    agg = agg_ref[0, :N_NODES, :] + agg_ref[1, :N_NODES, :]
    cnt = cnt_ref[0, :N_NODES, :] + cnt_ref[1, :N_NODES, :]
    rcp = 1.0 / jnp.clip(cnt[:, 0:1], 1.0, None)
    return agg * rcp


def _matT(a, w):
    return lax.dot_general(a, w, (((1,), (1,)), ((), ())),
                           preferred_element_type=jnp.float32)


def _bn(y, g_ref, be_ref):
    mu = jnp.mean(y, axis=0, keepdims=True)
    var = jnp.mean((y - mu) * (y - mu), axis=0, keepdims=True)
    return (y - mu) * lax.rsqrt(var + EPS) * g_ref[...] + be_ref[...]


def _tc_layer_body(residual, agg_ref, cnt_ref, x_ref, wl_ref, bl_ref, wr_ref,
                   g_ref, be_ref, o_ref):
    agg = _mean_div(agg_ref, cnt_ref)
    y = _matT(agg, wl_ref[...]) + bl_ref[...] + _matT(x_ref[...], wr_ref[...])
    y = jnp.maximum(y, 0.0)
    if residual:
        y = y + x_ref[...]
    o_ref[...] = _bn(y, g_ref, be_ref)


def _tc_head_body(agg_ref, cnt_ref, x_ref, wl_ref, bl_ref, wr_ref,
                  wfc_ref, bfc_ref, wfc1_ref, bfc1_ref, wfc2_ref, bfc2_ref,
                  g4_ref, be4_ref, g5_ref, be5_ref, o_ref):
    agg = _mean_div(agg_ref, cnt_ref)
    y = _matT(agg, wl_ref[...]) + bl_ref[...] + _matT(x_ref[...], wr_ref[...])
    x4 = jnp.maximum(y, 0.0) + x_ref[...]
    h = jnp.maximum(_matT(x4, wfc_ref[...]) + bfc_ref[...], 0.0)
    h = _bn(h, g4_ref, be4_ref)
    h = jnp.maximum(_matT(h, wfc1_ref[...]) + bfc1_ref[...], 0.0)
    h = _bn(h, g5_ref, be5_ref)
    o_ref[...] = _matT(h, wfc2_ref[...]) + bfc2_ref[...]


def _vmem_call(body, n_in, out_shape, name):
    return pl.pallas_call(
        body,
        in_specs=[pl.BlockSpec(memory_space=pltpu.VMEM)] * n_in,
        out_specs=pl.BlockSpec(memory_space=pltpu.VMEM),
        out_shape=out_shape,
        name=name,
    )


def kernel(x, edge_index, edge_weight, W1l, b1l, W1r, W2l, b2l, W2r,
           W3l, b3l, W3r, W4l, b4l, W4r, Wfc, bfc, Wfc1, bfc1, Wfc2, bfc2,
           g1, be1, g2, be2, g3, be3, g4, be4, g5, be5):
    npad = E_TOT - N_EDGES
    src3 = jnp.concatenate(
        [edge_index[0].astype(jnp.int32),
         jnp.zeros((npad,), jnp.int32)]).reshape(TOTC, E_CHK)
    dst = jnp.concatenate(
        [edge_index[1].astype(jnp.int32),
         jnp.full((npad,), N_PAD - 1, jnp.int32)]).reshape(TOTC, E_CHK)
    w3 = jnp.concatenate(
        [edge_weight.astype(jnp.float32),
         jnp.zeros((npad,), jnp.float32)]).reshape(TOTC, E_CHK)

    def row2(v):
        return v.reshape(1, -1)

    act = jax.ShapeDtypeStruct((N_NODES, D), jnp.float32)
    out40 = jax.ShapeDtypeStruct((N_NODES, 40), jnp.float32)

    tc_layer_res0 = _vmem_call(functools.partial(_tc_layer_body, False), 8,
                               act, "tc_layer_res0")
    tc_layer_res1 = _vmem_call(functools.partial(_tc_layer_body, True), 8,
                               act, "tc_layer_res1")
    tc_head = _vmem_call(_tc_head_body, 16, out40, "tc_head")

    cnt = _sc_cnt(dst)
    agg1 = _sc_agg(x, src3, w3, dst)
    x1 = tc_layer_res0(agg1, cnt, x, W1l, row2(b1l), W1r, row2(g1), row2(be1))

    agg2 = _sc_agg(x1, src3, w3, dst)
    x2 = tc_layer_res1(agg2, cnt, x1, W2l, row2(b2l), W2r, row2(g2), row2(be2))

    agg3 = _sc_agg(x2, src3, w3, dst)
    x3 = tc_layer_res1(agg3, cnt, x2, W3l, row2(b3l), W3r, row2(g3), row2(be3))

    agg4 = _sc_agg(x3, src3, w3, dst)
    out = tc_head(agg4, cnt, x3, W4l, row2(b4l), W4r,
                  Wfc, row2(bfc), Wfc1, row2(bfc1), Wfc2, row2(bfc2),
                  row2(g4), row2(be4), row2(g5), row2(be5))
    return out


# speedup vs baseline: 1.1717x; 1.1717x over previous
import functools

import jax
import jax.numpy as jnp
from jax import lax
from jax.experimental import pallas as pl
from jax.experimental.pallas import tpu as pltpu
from jax.experimental.pallas import tpu_sc as plsc

N_NODES = 10000
N_PAD = 10240
N_EDGES = 320000
D = 128
EPS = 1e-5

NC = 2
NS = 16
NW = NC * NS
E_CHK = 80
N_CHK = 128
TOTC = NW * N_CHK
E_TOT = TOTC * E_CHK
K0 = 256
K1 = 0
ROWS_PER_SUB = N_PAD // NS
NBUF = 4
NDST = 8


def _fill_vmem(ref, nrows, value):
    def body(i, _):
        for j in range(D // 16):
            ref[i, pl.ds(j * 16, 16)] = jnp.full((16,), value, jnp.float32)
        return 0
    lax.fori_loop(0, nrows, body, 0)


def _zero_share(slab, sid, shared):
    def zero_slab(k, _):
        r0 = sid * ROWS_PER_SUB + k * E_CHK
        pltpu.sync_copy(slab, shared.at[pl.ds(r0, E_CHK)])
        return 0
    lax.fori_loop(0, ROWS_PER_SUB // E_CHK, zero_slab, 0)


def _copy_share_out(slab, cid, sid, shared, out_hbm):
    def copy_slab(k, _):
        r0 = sid * ROWS_PER_SUB + k * E_CHK
        pltpu.sync_copy(shared.at[pl.ds(r0, E_CHK)], slab)
        pltpu.sync_copy(slab, out_hbm.at[cid, pl.ds(r0, E_CHK)])
        return 0
    lax.fori_loop(0, ROWS_PER_SUB // E_CHK, copy_slab, 0)


def _sc_agg_body(x_hbm, src_hbm, w_hbm, dst_hbm, agg_out,
                 ssliv_v, wsliv_v, dsliv_v, rows_v, acc_sh,
                 gsem, ssem, lsem, dsem):
    cid = lax.axis_index("c")
    sid = lax.axis_index("s")
    nchk = jnp.where(cid == 0, K0, K1)
    base = jnp.where(cid == 0, sid * K0, NS * K0 + sid * K1)

    _fill_vmem(rows_v.at[0], E_CHK, 0.0)
    _zero_share(rows_v.at[0], sid, acc_sh)
    plsc.subcore_barrier()

    def issue_sw(c, slot):
        pltpu.async_copy(src_hbm.at[base + c], ssliv_v.at[slot],
                         lsem.at[slot])
        pltpu.async_copy(w_hbm.at[base + c], wsliv_v.at[slot], lsem.at[slot])

    def issue_dst(c, slot):
        pltpu.async_copy(dst_hbm.at[base + c], dsliv_v.at[slot],
                         dsem.at[slot])

    def wait_sw(c, slot):
        pltpu.make_async_copy(src_hbm.at[base + c], ssliv_v.at[slot],
                              lsem.at[slot]).wait()
        pltpu.make_async_copy(w_hbm.at[base + c], wsliv_v.at[slot],
                              lsem.at[slot]).wait()

    def wait_dst(c, slot):
        pltpu.make_async_copy(dst_hbm.at[base + c], dsliv_v.at[slot],
                              dsem.at[slot]).wait()

    def issue_gather(c, slot):
        pltpu.async_copy(x_hbm.at[ssliv_v.at[slot]], rows_v.at[slot],
                         gsem.at[slot])

    def wait_gather(c, slot):
        pltpu.make_async_copy(x_hbm.at[ssliv_v.at[slot]],
                              rows_v.at[slot], gsem.at[slot]).wait()

    def issue_scatter(c, slot, dslot):
        pltpu.async_copy(rows_v.at[slot], acc_sh.at[dsliv_v.at[dslot]],
                         ssem.at[slot], add=True)

    def wait_scatter(c, slot, dslot):
        pltpu.make_async_copy(rows_v.at[slot], acc_sh.at[dsliv_v.at[dslot]],
                              ssem.at[slot]).wait()

    @pl.when(nchk > 0)
    def _():
        for k in range(NDST - 2):
            issue_dst(k, k)
        for k in range(NBUF):
            issue_sw(k, k)
        for k in range(2):
            wait_sw(k, k)
            issue_gather(k, k)

    def outer(cc, _):
        for u in range(NDST):
            c = cc * NDST + u
            r = u % NBUF

            @pl.when(c >= 2)
            def _():
                wait_scatter(c - 2, (u - 2) % NBUF, (u - 2) % NDST)

            @pl.when(c + NDST - 2 < nchk)
            def _():
                issue_dst(c + NDST - 2, (u - 2) % NDST)

            @pl.when(c + 2 < nchk)
            def _():
                wait_sw(c + 2, (u + 2) % NBUF)
                issue_gather(c + 2, (u + 2) % NBUF)

            wait_gather(c, r)

            def scale(k, _):
                ww = wsliv_v[r, pl.ds(k * 16, 16)]
                for l in range(16):
                    w = ww[l]
                    e = k * 16 + l
                    for j in range(D // 16):
                        v = rows_v[r, e, pl.ds(j * 16, 16)]
                        rows_v[r, e, pl.ds(j * 16, 16)] = v * w
                return 0
            lax.fori_loop(0, E_CHK // 16, scale, 0)

            @pl.when(c + NBUF < nchk)
            def _():
                issue_sw(c + NBUF, r)

            wait_dst(c, u)
            issue_scatter(c, r, u)
        return 0
    lax.fori_loop(0, nchk // NDST, outer, 0)

    @pl.when(nchk >= 2)
    def _():
        wait_scatter(nchk - 2, 2, NDST - 2)
        wait_scatter(nchk - 1, 3, NDST - 1)
    plsc.subcore_barrier()

    _copy_share_out(rows_v.at[0], cid, sid, acc_sh, agg_out)


def _make_sc_agg():
    mesh = plsc.VectorSubcoreMesh(core_axis_name="c", subcore_axis_name="s")
    scratch = [
        pltpu.VMEM((NBUF, E_CHK), jnp.int32),
        pltpu.VMEM((NBUF, E_CHK), jnp.float32),
        pltpu.VMEM((NDST, E_CHK), jnp.int32),
        pltpu.VMEM((NBUF, E_CHK, D), jnp.float32),
        pltpu.VMEM_SHARED((N_PAD, D), jnp.float32),
        pltpu.SemaphoreType.DMA((NBUF,)),
        pltpu.SemaphoreType.DMA((NBUF,)),
        pltpu.SemaphoreType.DMA((NBUF,)),
        pltpu.SemaphoreType.DMA((NDST,)),
    ]
    return pl.kernel(
        _sc_agg_body,
        out_type=jax.ShapeDtypeStruct((NC, N_PAD, D), jnp.float32),
        mesh=mesh, scratch_types=scratch, name="sc_agg")


def _sc_cnt_body(dst_hbm, cnt_out, dst_v, ones_v, zbuf_v, cnt_sh):
    cid = lax.axis_index("c")
    sid = lax.axis_index("s")
    wid = cid * NS + sid

    _fill_vmem(zbuf_v, E_CHK, 0.0)
    _fill_vmem(ones_v, E_CHK, 1.0)
    _zero_share(zbuf_v, sid, cnt_sh)
    plsc.subcore_barrier()

    pltpu.sync_copy(dst_hbm.at[pl.ds(wid * N_CHK, N_CHK)], dst_v)

    def chunk(c, _):
        pltpu.sync_copy(ones_v, cnt_sh.at[dst_v.at[c]], add=True)
        return 0
    lax.fori_loop(0, N_CHK, chunk, 0)
    plsc.subcore_barrier()

    _copy_share_out(zbuf_v, cid, sid, cnt_sh, cnt_out)


def _make_sc_cnt():
    mesh = plsc.VectorSubcoreMesh(core_axis_name="c", subcore_axis_name="s")
    scratch = [
        pltpu.VMEM((N_CHK, E_CHK), jnp.int32),
        pltpu.VMEM((E_CHK, D), jnp.float32),
        pltpu.VMEM((E_CHK, D), jnp.float32),
        pltpu.VMEM_SHARED((N_PAD, D), jnp.float32),
    ]
    return pl.kernel(
        _sc_cnt_body,
        out_type=jax.ShapeDtypeStruct((NC, N_PAD, D), jnp.float32),
        mesh=mesh, scratch_types=scratch, name="sc_cnt")


_sc_agg = _make_sc_agg()
_sc_cnt = _make_sc_cnt()


def _mean_div(agg_ref, cnt_ref):
    agg = agg_ref[0, :N_NODES, :] + agg_ref[1, :N_NODES, :]
    cnt = cnt_ref[0, :N_NODES, :] + cnt_ref[1, :N_NODES, :]
    rcp = 1.0 / jnp.clip(cnt[:, 0:1], 1.0, None)
    return agg * rcp


def _matT(a, w):
    return lax.dot_general(a, w, (((1,), (1,)), ((), ())),
                           preferred_element_type=jnp.float32)


def _bn(y, g_ref, be_ref):
    mu = jnp.mean(y, axis=0, keepdims=True)
    var = jnp.mean((y - mu) * (y - mu), axis=0, keepdims=True)
    return (y - mu) * lax.rsqrt(var + EPS) * g_ref[...] + be_ref[...]


def _tc_layer_body(residual, agg_ref, cnt_ref, x_ref, wl_ref, bl_ref, wr_ref,
                   g_ref, be_ref, o_ref):
    agg = _mean_div(agg_ref, cnt_ref)
    y = _matT(agg, wl_ref[...]) + bl_ref[...] + _matT(x_ref[...], wr_ref[...])
    y = jnp.maximum(y, 0.0)
    if residual:
        y = y + x_ref[...]
    o_ref[...] = _bn(y, g_ref, be_ref)


def _tc_head_body(agg_ref, cnt_ref, x_ref, wl_ref, bl_ref, wr_ref,
                  wfc_ref, bfc_ref, wfc1_ref, bfc1_ref, wfc2_ref, bfc2_ref,
                  g4_ref, be4_ref, g5_ref, be5_ref, o_ref):
    agg = _mean_div(agg_ref, cnt_ref)
    y = _matT(agg, wl_ref[...]) + bl_ref[...] + _matT(x_ref[...], wr_ref[...])
    x4 = jnp.maximum(y, 0.0) + x_ref[...]
    h = jnp.maximum(_matT(x4, wfc_ref[...]) + bfc_ref[...], 0.0)
    h = _bn(h, g4_ref, be4_ref)
    h = jnp.maximum(_matT(h, wfc1_ref[...]) + bfc1_ref[...], 0.0)
    h = _bn(h, g5_ref, be5_ref)
    o_ref[...] = _matT(h, wfc2_ref[...]) + bfc2_ref[...]


def _vmem_call(body, n_in, out_shape, name):
    return pl.pallas_call(
        body,
        in_specs=[pl.BlockSpec(memory_space=pltpu.VMEM)] * n_in,
        out_specs=pl.BlockSpec(memory_space=pltpu.VMEM),
        out_shape=out_shape,
        name=name,
    )


def kernel(x, edge_index, edge_weight, W1l, b1l, W1r, W2l, b2l, W2r,
           W3l, b3l, W3r, W4l, b4l, W4r, Wfc, bfc, Wfc1, bfc1, Wfc2, bfc2,
           g1, be1, g2, be2, g3, be3, g4, be4, g5, be5):
    npad = E_TOT - N_EDGES
    src3 = jnp.concatenate(
        [edge_index[0].astype(jnp.int32),
         jnp.zeros((npad,), jnp.int32)]).reshape(TOTC, E_CHK)
    dst = jnp.concatenate(
        [edge_index[1].astype(jnp.int32),
         jnp.full((npad,), N_PAD - 1, jnp.int32)]).reshape(TOTC, E_CHK)
    w3 = jnp.concatenate(
        [edge_weight.astype(jnp.float32),
         jnp.zeros((npad,), jnp.float32)]).reshape(TOTC, E_CHK)

    def row2(v):
        return v.reshape(1, -1)

    act = jax.ShapeDtypeStruct((N_NODES, D), jnp.float32)
    out40 = jax.ShapeDtypeStruct((N_NODES, 40), jnp.float32)

    tc_layer_res0 = _vmem_call(functools.partial(_tc_layer_body, False), 8,
                               act, "tc_layer_res0")
    tc_layer_res1 = _vmem_call(functools.partial(_tc_layer_body, True), 8,
                               act, "tc_layer_res1")
    tc_head = _vmem_call(_tc_head_body, 16, out40, "tc_head")

    cnt = _sc_cnt(dst)
    agg1 = _sc_agg(x, src3, w3, dst)
    x1 = tc_layer_res0(agg1, cnt, x, W1l, row2(b1l), W1r, row2(g1), row2(be1))

    agg2 = _sc_agg(x1, src3, w3, dst)
    x2 = tc_layer_res1(agg2, cnt, x1, W2l, row2(b2l), W2r, row2(g2), row2(be2))

    agg3 = _sc_agg(x2, src3, w3, dst)
    x3 = tc_layer_res1(agg3, cnt, x2, W3l, row2(b3l), W3r, row2(g3), row2(be3))

    agg4 = _sc_agg(x3, src3, w3, dst)
    out = tc_head(agg4, cnt, x3, W4l, row2(b4l), W4r,
                  Wfc, row2(bfc), Wfc1, row2(bfc1), Wfc2, row2(bfc2),
                  row2(g4), row2(be4), row2(g5), row2(be5))
    return out


# speedup vs baseline: 3.9152x; 3.3416x over previous
import functools

import jax
import jax.numpy as jnp
from jax import lax
from jax.experimental import pallas as pl
from jax.experimental.pallas import tpu as pltpu
from jax.experimental.pallas import tpu_sc as plsc

N_NODES = 10000
N_PAD = 10240
N_EDGES = 320000
D = 128
EPS = 1e-5

NC = 2
NS = 16
NW = NC * NS
E_CHK = 80
N_CHK = 128
TOTC = NW * N_CHK
E_TOT = TOTC * E_CHK
K0 = 128
K1 = 128
ROWS_PER_SUB = N_PAD // NS
NBUF = 4
NDST = 8


def _fill_vmem(ref, nrows, value):
    def body(i, _):
        for j in range(D // 16):
            ref[i, pl.ds(j * 16, 16)] = jnp.full((16,), value, jnp.float32)
        return 0
    lax.fori_loop(0, nrows, body, 0)


def _zero_share(slab, sid, shared):
    def zero_slab(k, _):
        r0 = sid * ROWS_PER_SUB + k * E_CHK
        pltpu.sync_copy(slab, shared.at[pl.ds(r0, E_CHK)])
        return 0
    lax.fori_loop(0, ROWS_PER_SUB // E_CHK, zero_slab, 0)


def _copy_share_out(slab, cid, sid, shared, out_hbm):
    def copy_slab(k, _):
        r0 = sid * ROWS_PER_SUB + k * E_CHK
        pltpu.sync_copy(shared.at[pl.ds(r0, E_CHK)], slab)
        pltpu.sync_copy(slab, out_hbm.at[cid, pl.ds(r0, E_CHK)])
        return 0
    lax.fori_loop(0, ROWS_PER_SUB // E_CHK, copy_slab, 0)


def _sc_agg_body(x_hbm, src_hbm, w_hbm, dst_hbm, agg_out,
                 ssliv_v, wsliv_v, dsliv_v, rows_v, acc_sh,
                 gsem, ssem, lsem, dsem):
    cid = lax.axis_index("c")
    sid = lax.axis_index("s")
    nchk = jnp.where(cid == 0, K0, K1)
    base = jnp.where(cid == 0, sid * K0, NS * K0 + sid * K1)

    _fill_vmem(rows_v.at[0], E_CHK, 0.0)
    _zero_share(rows_v.at[0], sid, acc_sh)
    plsc.subcore_barrier()

    def issue_sw(c, slot):
        pltpu.async_copy(src_hbm.at[base + c], ssliv_v.at[slot],
                         lsem.at[slot])
        pltpu.async_copy(w_hbm.at[base + c], wsliv_v.at[slot], lsem.at[slot])

    def issue_dst(c, slot):
        pltpu.async_copy(dst_hbm.at[base + c], dsliv_v.at[slot],
                         dsem.at[slot])

    def wait_sw(c, slot):
        pltpu.make_async_copy(src_hbm.at[base + c], ssliv_v.at[slot],
                              lsem.at[slot]).wait()
        pltpu.make_async_copy(w_hbm.at[base + c], wsliv_v.at[slot],
                              lsem.at[slot]).wait()

    def wait_dst(c, slot):
        pltpu.make_async_copy(dst_hbm.at[base + c], dsliv_v.at[slot],
                              dsem.at[slot]).wait()

    def issue_gather(c, slot):
        pltpu.async_copy(x_hbm.at[ssliv_v.at[slot]], rows_v.at[slot],
                         gsem.at[slot])

    def wait_gather(c, slot):
        pltpu.make_async_copy(x_hbm.at[ssliv_v.at[slot]],
                              rows_v.at[slot], gsem.at[slot]).wait()

    def issue_scatter(c, slot, dslot):
        pltpu.async_copy(rows_v.at[slot], acc_sh.at[dsliv_v.at[dslot]],
                         ssem.at[slot], add=True)

    def wait_scatter(c, slot, dslot):
        pltpu.make_async_copy(rows_v.at[slot], acc_sh.at[dsliv_v.at[dslot]],
                              ssem.at[slot]).wait()

    @pl.when(nchk > 0)
    def _():
        for k in range(NDST - 2):
            issue_dst(k, k)
        for k in range(NBUF):
            issue_sw(k, k)
        for k in range(2):
            wait_sw(k, k)
            issue_gather(k, k)

    def outer(cc, _):
        for u in range(NDST):
            c = cc * NDST + u
            r = u % NBUF

            @pl.when(c >= 2)
            def _():
                wait_scatter(c - 2, (u - 2) % NBUF, (u - 2) % NDST)

            @pl.when(c + NDST - 2 < nchk)
            def _():
                issue_dst(c + NDST - 2, (u - 2) % NDST)

            @pl.when(c + 2 < nchk)
            def _():
                wait_sw(c + 2, (u + 2) % NBUF)
                issue_gather(c + 2, (u + 2) % NBUF)

            wait_gather(c, r)

            def scale(k, _):
                ww = wsliv_v[r, pl.ds(k * 16, 16)]
                for l in range(16):
                    w = ww[l]
                    e = k * 16 + l
                    for j in range(D // 16):
                        v = rows_v[r, e, pl.ds(j * 16, 16)]
                        rows_v[r, e, pl.ds(j * 16, 16)] = v * w
                return 0
            lax.fori_loop(0, E_CHK // 16, scale, 0)

            @pl.when(c + NBUF < nchk)
            def _():
                issue_sw(c + NBUF, r)

            wait_dst(c, u)
            issue_scatter(c, r, u)
        return 0
    lax.fori_loop(0, nchk // NDST, outer, 0)

    @pl.when(nchk >= 2)
    def _():
        wait_scatter(nchk - 2, 2, NDST - 2)
        wait_scatter(nchk - 1, 3, NDST - 1)
    plsc.subcore_barrier()

    _copy_share_out(rows_v.at[0], cid, sid, acc_sh, agg_out)


def _make_sc_agg():
    mesh = plsc.VectorSubcoreMesh(core_axis_name="c", subcore_axis_name="s")
    scratch = [
        pltpu.VMEM((NBUF, E_CHK), jnp.int32),
        pltpu.VMEM((NBUF, E_CHK), jnp.float32),
        pltpu.VMEM((NDST, E_CHK), jnp.int32),
        pltpu.VMEM((NBUF, E_CHK, D), jnp.float32),
        pltpu.VMEM_SHARED((N_PAD, D), jnp.float32),
        pltpu.SemaphoreType.DMA((NBUF,)),
        pltpu.SemaphoreType.DMA((NBUF,)),
        pltpu.SemaphoreType.DMA((NBUF,)),
        pltpu.SemaphoreType.DMA((NDST,)),
    ]
    return pl.kernel(
        _sc_agg_body,
        out_type=jax.ShapeDtypeStruct((NC, N_PAD, D), jnp.float32),
        mesh=mesh, scratch_types=scratch, name="sc_agg")


def _sc_cnt_body(dst_hbm, cnt_out, dst_v, ones_v, zbuf_v, cnt_sh):
    cid = lax.axis_index("c")
    sid = lax.axis_index("s")
    wid = cid * NS + sid

    _fill_vmem(zbuf_v, E_CHK, 0.0)
    _fill_vmem(ones_v, E_CHK, 1.0)
    _zero_share(zbuf_v, sid, cnt_sh)
    plsc.subcore_barrier()

    pltpu.sync_copy(dst_hbm.at[pl.ds(wid * N_CHK, N_CHK)], dst_v)

    def chunk(c, _):
        pltpu.sync_copy(ones_v, cnt_sh.at[dst_v.at[c]], add=True)
        return 0
    lax.fori_loop(0, N_CHK, chunk, 0)
    plsc.subcore_barrier()

    _copy_share_out(zbuf_v, cid, sid, cnt_sh, cnt_out)


def _make_sc_cnt():
    mesh = plsc.VectorSubcoreMesh(core_axis_name="c", subcore_axis_name="s")
    scratch = [
        pltpu.VMEM((N_CHK, E_CHK), jnp.int32),
        pltpu.VMEM((E_CHK, D), jnp.float32),
        pltpu.VMEM((E_CHK, D), jnp.float32),
        pltpu.VMEM_SHARED((N_PAD, D), jnp.float32),
    ]
    return pl.kernel(
        _sc_cnt_body,
        out_type=jax.ShapeDtypeStruct((NC, N_PAD, D), jnp.float32),
        mesh=mesh, scratch_types=scratch, name="sc_cnt")


_sc_agg = _make_sc_agg()
_sc_cnt = _make_sc_cnt()


def _mean_div(agg_ref, cnt_ref):
    agg = agg_ref[0, :N_NODES, :] + agg_ref[1, :N_NODES, :]
    cnt = cnt_ref[0, :N_NODES, :] + cnt_ref[1, :N_NODES, :]
    rcp = 1.0 / jnp.clip(cnt[:, 0:1], 1.0, None)
    return agg * rcp


def _matT(a, w):
    return lax.dot_general(a, w, (((1,), (1,)), ((), ())),
                           preferred_element_type=jnp.float32)


def _bn(y, g_ref, be_ref):
    mu = jnp.mean(y, axis=0, keepdims=True)
    var = jnp.mean((y - mu) * (y - mu), axis=0, keepdims=True)
    return (y - mu) * lax.rsqrt(var + EPS) * g_ref[...] + be_ref[...]


def _tc_layer_body(residual, agg_ref, cnt_ref, x_ref, wl_ref, bl_ref, wr_ref,
                   g_ref, be_ref, o_ref):
    agg = _mean_div(agg_ref, cnt_ref)
    y = _matT(agg, wl_ref[...]) + bl_ref[...] + _matT(x_ref[...], wr_ref[...])
    y = jnp.maximum(y, 0.0)
    if residual:
        y = y + x_ref[...]
    o_ref[...] = _bn(y, g_ref, be_ref)


def _tc_head_body(agg_ref, cnt_ref, x_ref, wl_ref, bl_ref, wr_ref,
                  wfc_ref, bfc_ref, wfc1_ref, bfc1_ref, wfc2_ref, bfc2_ref,
                  g4_ref, be4_ref, g5_ref, be5_ref, o_ref):
    agg = _mean_div(agg_ref, cnt_ref)
    y = _matT(agg, wl_ref[...]) + bl_ref[...] + _matT(x_ref[...], wr_ref[...])
    x4 = jnp.maximum(y, 0.0) + x_ref[...]
    h = jnp.maximum(_matT(x4, wfc_ref[...]) + bfc_ref[...], 0.0)
    h = _bn(h, g4_ref, be4_ref)
    h = jnp.maximum(_matT(h, wfc1_ref[...]) + bfc1_ref[...], 0.0)
    h = _bn(h, g5_ref, be5_ref)
    o_ref[...] = _matT(h, wfc2_ref[...]) + bfc2_ref[...]


def _vmem_call(body, n_in, out_shape, name):
    return pl.pallas_call(
        body,
        in_specs=[pl.BlockSpec(memory_space=pltpu.VMEM)] * n_in,
        out_specs=pl.BlockSpec(memory_space=pltpu.VMEM),
        out_shape=out_shape,
        name=name,
    )


def kernel(x, edge_index, edge_weight, W1l, b1l, W1r, W2l, b2l, W2r,
           W3l, b3l, W3r, W4l, b4l, W4r, Wfc, bfc, Wfc1, bfc1, Wfc2, bfc2,
           g1, be1, g2, be2, g3, be3, g4, be4, g5, be5):
    npad = E_TOT - N_EDGES
    pad_iota = jnp.arange(npad, dtype=jnp.int32)
    src3 = jnp.concatenate(
        [edge_index[0].astype(jnp.int32),
         pad_iota % N_NODES]).reshape(TOTC, E_CHK)
    dst = jnp.concatenate(
        [edge_index[1].astype(jnp.int32),
         N_NODES + pad_iota % (N_PAD - N_NODES)]).reshape(TOTC, E_CHK)
    w3 = jnp.concatenate(
        [edge_weight.astype(jnp.float32),
         jnp.zeros((npad,), jnp.float32)]).reshape(TOTC, E_CHK)

    def row2(v):
        return v.reshape(1, -1)

    act = jax.ShapeDtypeStruct((N_NODES, D), jnp.float32)
    out40 = jax.ShapeDtypeStruct((N_NODES, 40), jnp.float32)

    tc_layer_res0 = _vmem_call(functools.partial(_tc_layer_body, False), 8,
                               act, "tc_layer_res0")
    tc_layer_res1 = _vmem_call(functools.partial(_tc_layer_body, True), 8,
                               act, "tc_layer_res1")
    tc_head = _vmem_call(_tc_head_body, 16, out40, "tc_head")

    cnt = _sc_cnt(dst)
    agg1 = _sc_agg(x, src3, w3, dst)
    x1 = tc_layer_res0(agg1, cnt, x, W1l, row2(b1l), W1r, row2(g1), row2(be1))

    agg2 = _sc_agg(x1, src3, w3, dst)
    x2 = tc_layer_res1(agg2, cnt, x1, W2l, row2(b2l), W2r, row2(g2), row2(be2))

    agg3 = _sc_agg(x2, src3, w3, dst)
    x3 = tc_layer_res1(agg3, cnt, x2, W3l, row2(b3l), W3r, row2(g3), row2(be3))

    agg4 = _sc_agg(x3, src3, w3, dst)
    out = tc_head(agg4, cnt, x3, W4l, row2(b4l), W4r,
                  Wfc, row2(bfc), Wfc1, row2(bfc1), Wfc2, row2(bfc2),
                  row2(g4), row2(be4), row2(g5), row2(be5))
    return out


# speedup vs baseline: 3.9202x; 1.0013x over previous
import functools

import jax
import jax.numpy as jnp
from jax import lax
from jax.experimental import pallas as pl
from jax.experimental.pallas import tpu as pltpu
from jax.experimental.pallas import tpu_sc as plsc

N_NODES = 10000
N_PAD = 10240
N_EDGES = 320000
D = 128
EPS = 1e-5

NC = 2
NS = 16
NW = NC * NS
E_CHK = 80
N_CHK = 128
TOTC = NW * N_CHK
E_TOT = TOTC * E_CHK
K0 = 128
K1 = 128
ROWS_PER_SUB = N_PAD // NS
NBUF = 4
NDST = 8


def _fill_vmem(ref, nrows, value):
    def body(i, _):
        for j in range(D // 16):
            ref[i, pl.ds(j * 16, 16)] = jnp.full((16,), value, jnp.float32)
        return 0
    lax.fori_loop(0, nrows, body, 0)


def _zero_share(slab, sid, shared):
    def zero_slab(k, _):
        r0 = sid * ROWS_PER_SUB + k * E_CHK
        pltpu.sync_copy(slab, shared.at[pl.ds(r0, E_CHK)])
        return 0
    lax.fori_loop(0, ROWS_PER_SUB // E_CHK, zero_slab, 0)


def _copy_share_out(slab, cid, sid, shared, out_hbm):
    def copy_slab(k, _):
        r0 = sid * ROWS_PER_SUB + k * E_CHK
        pltpu.sync_copy(shared.at[pl.ds(r0, E_CHK)], slab)
        pltpu.sync_copy(slab, out_hbm.at[cid, pl.ds(r0, E_CHK)])
        return 0
    lax.fori_loop(0, ROWS_PER_SUB // E_CHK, copy_slab, 0)


def _sc_agg_body(x_hbm, src_hbm, w_hbm, dst_hbm, agg_out,
                 ssliv_v, wsliv_v, dsliv_v, rows_v, acc_sh,
                 gsem, ssem, lsem, dsem):
    cid = lax.axis_index("c")
    sid = lax.axis_index("s")
    nchk = jnp.where(cid == 0, K0, K1)
    base = jnp.where(cid == 0, sid * K0, NS * K0 + sid * K1)

    _fill_vmem(rows_v.at[0], E_CHK, 0.0)
    _zero_share(rows_v.at[0], sid, acc_sh)
    plsc.subcore_barrier()

    def issue_sw(c, slot):
        pltpu.async_copy(src_hbm.at[base + c], ssliv_v.at[slot],
                         lsem.at[slot])
        pltpu.async_copy(w_hbm.at[base + c], wsliv_v.at[slot], lsem.at[slot])

    def issue_dst(c, slot):
        pltpu.async_copy(dst_hbm.at[base + c], dsliv_v.at[slot],
                         dsem.at[slot])

    def wait_sw(c, slot):
        pltpu.make_async_copy(src_hbm.at[base + c], ssliv_v.at[slot],
                              lsem.at[slot]).wait()
        pltpu.make_async_copy(w_hbm.at[base + c], wsliv_v.at[slot],
                              lsem.at[slot]).wait()

    def wait_dst(c, slot):
        pltpu.make_async_copy(dst_hbm.at[base + c], dsliv_v.at[slot],
                              dsem.at[slot]).wait()

    def issue_gather(c, slot):
        pltpu.async_copy(x_hbm.at[ssliv_v.at[slot]], rows_v.at[slot],
                         gsem.at[slot])

    def wait_gather(c, slot):
        pltpu.make_async_copy(x_hbm.at[ssliv_v.at[slot]],
                              rows_v.at[slot], gsem.at[slot]).wait()

    def issue_scatter(c, slot, dslot):
        pltpu.async_copy(rows_v.at[slot], acc_sh.at[dsliv_v.at[dslot]],
                         ssem.at[slot], add=True)

    def wait_scatter(c, slot, dslot):
        pltpu.make_async_copy(rows_v.at[slot], acc_sh.at[dsliv_v.at[dslot]],
                              ssem.at[slot]).wait()

    @pl.when(nchk > 0)
    def _():
        for k in range(NDST - 2):
            issue_dst(k, k)
        for k in range(NBUF):
            issue_sw(k, k)
        for k in range(2):
            wait_sw(k, k)
            issue_gather(k, k)

    def outer(cc, _):
        for u in range(NDST):
            c = cc * NDST + u
            r = u % NBUF

            @pl.when(c >= 2)
            def _():
                wait_scatter(c - 2, (u - 2) % NBUF, (u - 2) % NDST)

            @pl.when(c + NDST - 2 < nchk)
            def _():
                issue_dst(c + NDST - 2, (u - 2) % NDST)

            @pl.when(c + 2 < nchk)
            def _():
                wait_sw(c + 2, (u + 2) % NBUF)
                issue_gather(c + 2, (u + 2) % NBUF)

            wait_gather(c, r)

            def scale(k, _):
                ww = wsliv_v[r, pl.ds(k * 16, 16)]
                for l in range(16):
                    w = ww[l]
                    e = k * 16 + l
                    for j in range(D // 16):
                        v = rows_v[r, e, pl.ds(j * 16, 16)]
                        rows_v[r, e, pl.ds(j * 16, 16)] = v * w
                return 0
            lax.fori_loop(0, E_CHK // 16, scale, 0)

            @pl.when(c + NBUF < nchk)
            def _():
                issue_sw(c + NBUF, r)

            wait_dst(c, u)
            issue_scatter(c, r, u)
        return 0
    lax.fori_loop(0, nchk // NDST, outer, 0)

    @pl.when(nchk >= 2)
    def _():
        wait_scatter(nchk - 2, 2, NDST - 2)
        wait_scatter(nchk - 1, 3, NDST - 1)
    plsc.subcore_barrier()

    _copy_share_out(rows_v.at[0], cid, sid, acc_sh, agg_out)


def _make_sc_agg():
    mesh = plsc.VectorSubcoreMesh(core_axis_name="c", subcore_axis_name="s")
    scratch = [
        pltpu.VMEM((NBUF, E_CHK), jnp.int32),
        pltpu.VMEM((NBUF, E_CHK), jnp.float32),
        pltpu.VMEM((NDST, E_CHK), jnp.int32),
        pltpu.VMEM((NBUF, E_CHK, D), jnp.float32),
        pltpu.VMEM_SHARED((N_PAD, D), jnp.float32),
        pltpu.SemaphoreType.DMA((NBUF,)),
        pltpu.SemaphoreType.DMA((NBUF,)),
        pltpu.SemaphoreType.DMA((NBUF,)),
        pltpu.SemaphoreType.DMA((NDST,)),
    ]
    return pl.kernel(
        _sc_agg_body,
        out_type=jax.ShapeDtypeStruct((NC, N_PAD, D), jnp.float32),
        mesh=mesh, scratch_types=scratch, name="sc_agg")


def _sc_cnt_body(dst_hbm, cnt_out, dst_v, ones_v, zbuf_v, cnt_sh, csem):
    cid = lax.axis_index("c")
    sid = lax.axis_index("s")
    wid = cid * NS + sid

    _fill_vmem(zbuf_v, E_CHK, 0.0)
    _fill_vmem(ones_v, E_CHK, 1.0)
    _zero_share(zbuf_v, sid, cnt_sh)
    plsc.subcore_barrier()

    pltpu.sync_copy(dst_hbm.at[pl.ds(wid * N_CHK, N_CHK)], dst_v)

    def chunk4(cc, _):
        for u in range(4):
            c = cc * 4 + u

            @pl.when(c >= 2)
            def _():
                pltpu.make_async_copy(
                    ones_v, cnt_sh.at[dst_v.at[c - 2]],
                    csem.at[(u - 2) % 4]).wait()
            pltpu.async_copy(ones_v, cnt_sh.at[dst_v.at[c]], csem.at[u],
                             add=True)
        return 0
    lax.fori_loop(0, N_CHK // 4, chunk4, 0)
    pltpu.make_async_copy(ones_v, cnt_sh.at[dst_v.at[N_CHK - 2]],
                          csem.at[2]).wait()
    pltpu.make_async_copy(ones_v, cnt_sh.at[dst_v.at[N_CHK - 1]],
                          csem.at[3]).wait()
    plsc.subcore_barrier()

    _copy_share_out(zbuf_v, cid, sid, cnt_sh, cnt_out)


def _make_sc_cnt():
    mesh = plsc.VectorSubcoreMesh(core_axis_name="c", subcore_axis_name="s")
    scratch = [
        pltpu.VMEM((N_CHK, E_CHK), jnp.int32),
        pltpu.VMEM((E_CHK, D), jnp.float32),
        pltpu.VMEM((E_CHK, D), jnp.float32),
        pltpu.VMEM_SHARED((N_PAD, D), jnp.float32),
        pltpu.SemaphoreType.DMA((4,)),
    ]
    return pl.kernel(
        _sc_cnt_body,
        out_type=jax.ShapeDtypeStruct((NC, N_PAD, D), jnp.float32),
        mesh=mesh, scratch_types=scratch, name="sc_cnt")


_sc_agg = _make_sc_agg()
_sc_cnt = _make_sc_cnt()


def _mean_div(agg_ref, cnt_ref):
    agg = agg_ref[0, :N_NODES, :] + agg_ref[1, :N_NODES, :]
    cnt = cnt_ref[0, :N_NODES, :] + cnt_ref[1, :N_NODES, :]
    rcp = 1.0 / jnp.clip(cnt[:, 0:1], 1.0, None)
    return agg * rcp


def _matT(a, w):
    return lax.dot_general(a, w, (((1,), (1,)), ((), ())),
                           preferred_element_type=jnp.float32)


def _bn(y, g_ref, be_ref):
    mu = jnp.mean(y, axis=0, keepdims=True)
    var = jnp.mean((y - mu) * (y - mu), axis=0, keepdims=True)
    return (y - mu) * lax.rsqrt(var + EPS) * g_ref[...] + be_ref[...]


def _tc_layer_body(residual, agg_ref, cnt_ref, x_ref, wl_ref, bl_ref, wr_ref,
                   g_ref, be_ref, o_ref):
    agg = _mean_div(agg_ref, cnt_ref)
    y = _matT(agg, wl_ref[...]) + bl_ref[...] + _matT(x_ref[...], wr_ref[...])
    y = jnp.maximum(y, 0.0)
    if residual:
        y = y + x_ref[...]
    o_ref[...] = _bn(y, g_ref, be_ref)


def _tc_head_body(agg_ref, cnt_ref, x_ref, wl_ref, bl_ref, wr_ref,
                  wfc_ref, bfc_ref, wfc1_ref, bfc1_ref, wfc2_ref, bfc2_ref,
                  g4_ref, be4_ref, g5_ref, be5_ref, o_ref):
    agg = _mean_div(agg_ref, cnt_ref)
    y = _matT(agg, wl_ref[...]) + bl_ref[...] + _matT(x_ref[...], wr_ref[...])
    x4 = jnp.maximum(y, 0.0) + x_ref[...]
    h = jnp.maximum(_matT(x4, wfc_ref[...]) + bfc_ref[...], 0.0)
    h = _bn(h, g4_ref, be4_ref)
    h = jnp.maximum(_matT(h, wfc1_ref[...]) + bfc1_ref[...], 0.0)
    h = _bn(h, g5_ref, be5_ref)
    o_ref[...] = _matT(h, wfc2_ref[...]) + bfc2_ref[...]


def _vmem_call(body, n_in, out_shape, name):
    return pl.pallas_call(
        body,
        in_specs=[pl.BlockSpec(memory_space=pltpu.VMEM)] * n_in,
        out_specs=pl.BlockSpec(memory_space=pltpu.VMEM),
        out_shape=out_shape,
        name=name,
    )


def kernel(x, edge_index, edge_weight, W1l, b1l, W1r, W2l, b2l, W2r,
           W3l, b3l, W3r, W4l, b4l, W4r, Wfc, bfc, Wfc1, bfc1, Wfc2, bfc2,
           g1, be1, g2, be2, g3, be3, g4, be4, g5, be5):
    npad = E_TOT - N_EDGES
    pad_iota = jnp.arange(npad, dtype=jnp.int32)
    src3 = jnp.concatenate(
        [edge_index[0].astype(jnp.int32),
         pad_iota % N_NODES]).reshape(TOTC, E_CHK)
    dst = jnp.concatenate(
        [edge_index[1].astype(jnp.int32),
         N_NODES + pad_iota % (N_PAD - N_NODES)]).reshape(TOTC, E_CHK)
    w3 = jnp.concatenate(
        [edge_weight.astype(jnp.float32),
         jnp.zeros((npad,), jnp.float32)]).reshape(TOTC, E_CHK)

    def row2(v):
        return v.reshape(1, -1)

    act = jax.ShapeDtypeStruct((N_NODES, D), jnp.float32)
    out40 = jax.ShapeDtypeStruct((N_NODES, 40), jnp.float32)

    tc_layer_res0 = _vmem_call(functools.partial(_tc_layer_body, False), 8,
                               act, "tc_layer_res0")
    tc_layer_res1 = _vmem_call(functools.partial(_tc_layer_body, True), 8,
                               act, "tc_layer_res1")
    tc_head = _vmem_call(_tc_head_body, 16, out40, "tc_head")

    cnt = _sc_cnt(dst)
    agg1 = _sc_agg(x, src3, w3, dst)
    x1 = tc_layer_res0(agg1, cnt, x, W1l, row2(b1l), W1r, row2(g1), row2(be1))

    agg2 = _sc_agg(x1, src3, w3, dst)
    x2 = tc_layer_res1(agg2, cnt, x1, W2l, row2(b2l), W2r, row2(g2), row2(be2))

    agg3 = _sc_agg(x2, src3, w3, dst)
    x3 = tc_layer_res1(agg3, cnt, x2, W3l, row2(b3l), W3r, row2(g3), row2(be3))

    agg4 = _sc_agg(x3, src3, w3, dst)
    out = tc_head(agg4, cnt, x3, W4l, row2(b4l), W4r,
                  Wfc, row2(bfc), Wfc1, row2(bfc1), Wfc2, row2(bfc2),
                  row2(g4), row2(be4), row2(g5), row2(be5))
    return out
